# baseline (speedup 1.0000x reference)
import functools

import jax
import jax.numpy as jnp
from jax import lax
from jax.experimental import pallas as pl
from jax.experimental.pallas import tpu as pltpu
from jax.experimental.pallas import tpu_sc as plsc

N = 10000
E = 320000
D = 128
K = 2
EPS = 1e-5

N_PAD = 10240
NW = 32
C = 40
EPW = 10080
NCH = EPW // C
E_PAD = NW * EPW
RPT = N_PAD // 16

_mesh = plsc.VectorSubcoreMesh(core_axis_name="c", subcore_axis_name="s")

_f32 = jnp.float32
_i32 = jnp.int32


@functools.partial(
    pl.kernel,
    mesh=_mesh,
    out_type=jax.ShapeDtypeStruct((2, N_PAD, D), _f32),
    scratch_types=[
        pltpu.VMEM((3, C), _i32),
        pltpu.VMEM((C, D), _f32),
        pltpu.VMEM((C, D), _f32),
        pltpu.VMEM((C, D), _f32),
        pltpu.VMEM_SHARED((N_PAD, D), _f32),
        pltpu.SemaphoreType.DMA,
    ],
)
def _sc_hist(idx3, hist_out, iv, ones0, ones1, ones2, hacc, sem):
    ones = (ones0, ones1, ones2)
    cid = lax.axis_index("c")
    sid = lax.axis_index("s")
    wid = sid * 2 + cid

    z16 = jnp.zeros((16,), _f32)

    def _fill(r, carry):
        for j in range(3):
            for v in range(8):
                ones[j][r, pl.ds(v * 16, 16)] = z16
        return carry

    lax.fori_loop(0, C, _fill, 0)

    def _zero_acc(q, carry):
        pltpu.sync_copy(ones0, hacc.at[pl.ds(sid * RPT + q * C, C)])
        return carry

    lax.fori_loop(0, RPT // C, _zero_acc, 0)

    for j in range(3):
        onerow = jnp.where(lax.iota(_i32, 16) == j, 1.0, 0.0).astype(_f32)

        def _fill_ones(r, carry, j=j, onerow=onerow):
            ones[j][r, pl.ds(0, 16)] = onerow
            return carry

        lax.fori_loop(0, C, _fill_ones, 0)

    plsc.subcore_barrier()

    def _chunk(t, carry):
        pltpu.sync_copy(idx3.at[wid, t], iv)
        for j in range(3):
            pltpu.sync_copy(ones[j], hacc.at[iv.at[j]], add=True)
        return carry

    lax.fori_loop(0, NCH, _chunk, 0)

    plsc.subcore_barrier()
    pltpu.sync_copy(hacc.at[pl.ds(sid * RPT, RPT)],
                    hist_out.at[cid, pl.ds(sid * RPT, RPT)])


@functools.partial(
    pl.kernel,
    mesh=_mesh,
    out_type=jax.ShapeDtypeStruct((NW, 3, D), _f32),
    scratch_types=[
        pltpu.VMEM((3, C), _i32),
        pltpu.VMEM((3, C), _i32),
    ] + [pltpu.VMEM((C, 3 * D), _f32) for _ in range(6)] + [
        pltpu.VMEM((3, D), _f32),
        pltpu.SemaphoreType.DMA,
        pltpu.SemaphoreType.DMA,
    ],
)
def _sc_pass_a(g0, g1, g2, idx3, ssq_out,
               iv_a, iv_b, ba0, ba1, ba2, bb0, bb1, bb2,
               ssq_v, sem_a, sem_b):
    tabs = (g0, g1, g2)
    ivs = (iv_a, iv_b)
    bufs = ((ba0, ba1, ba2), (bb0, bb1, bb2))
    sems = (sem_a, sem_b)
    cid = lax.axis_index("c")
    sid = lax.axis_index("s")
    wid = sid * 2 + cid

    z16 = jnp.zeros((16,), _f32)

    def _fire(t, s):
        pltpu.sync_copy(idx3.at[wid, t], ivs[s])
        for j in range(3):
            pltpu.async_copy(tabs[j].at[ivs[s].at[j]], bufs[s][j], sems[s])

    def _drain(s):
        for j in range(3):
            pltpu.make_async_copy(tabs[j].at[ivs[s].at[j]], bufs[s][j],
                                  sems[s]).wait()

    def _consume(s, acc):
        bj = bufs[s]

        def _row(r, a):
            out = []
            for c in range(3):
                for v in range(8):
                    sl = pl.ds(c * D + v * 16, 16)
                    y = bj[0][r, sl] + bj[1][r, sl] + bj[2][r, sl]
                    out.append(a[c * 8 + v] + y * y)
            return tuple(out)

        return plsc.parallel_loop(0, C, unroll=2, carry=acc)(_row)

    _fire(0, 0)
    init = tuple(z16 for _ in range(24))

    def _pair(p, acc):
        for par in (0, 1):
            t = 2 * p + par

            @pl.when(t + 1 < NCH)
            def _():
                _fire(t + 1, 1 - par)

            _drain(par)
            acc = _consume(par, acc)
        return acc

    acc = lax.fori_loop(0, NCH // 2, _pair, init)
    for c in range(3):
        for v in range(8):
            ssq_v[c, pl.ds(v * 16, 16)] = acc[c * 8 + v]
    pltpu.sync_copy(ssq_v, ssq_out.at[wid])


@functools.partial(
    pl.kernel,
    mesh=_mesh,
    out_type=jax.ShapeDtypeStruct((2, 3, N_PAD, D), _f32),
    scratch_types=[
        pltpu.VMEM((3, C), _i32),
        pltpu.VMEM((3, C), _i32),
    ] + [pltpu.VMEM((C, D), _f32) for _ in range(6)] + [
        pltpu.VMEM((C, D), _f32),
        pltpu.VMEM((C, D), _f32),
        pltpu.VMEM((3, 2, D), _f32),
        pltpu.VMEM_SHARED((N_PAD, D), _f32),
        pltpu.SemaphoreType.DMA,
        pltpu.SemaphoreType.DMA,
        pltpu.SemaphoreType.DMA,
    ],
)
def _sc_pass_b(t0, t1, t2, t3, t4, t5, t6, t7, t8, idx3, ss,
               s_out,
               iv_a, iv_b, ba0, ba1, ba2, bb0, bb1, bb2,
               zb0, zb1, ss_v, sacc, sem_a, sem_b, sem_sc):
    tables = (t0, t1, t2, t3, t4, t5, t6, t7, t8)
    ivs = (iv_a, iv_b)
    bufs = ((ba0, ba1, ba2), (bb0, bb1, bb2))
    zbs = (zb0, zb1)
    sems = (sem_a, sem_b)
    cid = lax.axis_index("c")
    sid = lax.axis_index("s")
    wid = sid * 2 + cid

    z16 = jnp.zeros((16,), _f32)

    pltpu.sync_copy(ss, ss_v)

    def _fill_zero(r, carry):
        for v in range(8):
            zb0[r, pl.ds(v * 16, 16)] = z16
        return carry

    lax.fori_loop(0, C, _fill_zero, 0)

    for c in range(3):
        scale_vs = [ss_v[c, 0, pl.ds(v * 16, 16)] for v in range(8)]
        shift_vs = [ss_v[c, 1, pl.ds(v * 16, 16)] for v in range(8)]

        if c > 0:
            lax.fori_loop(0, C, _fill_zero, 0)

        def _zero_acc(q, carry):
            pltpu.sync_copy(zb0, sacc.at[pl.ds(sid * RPT + q * C, C)])
            return carry

        lax.fori_loop(0, RPT // C, _zero_acc, 0)
        plsc.subcore_barrier()

        def _fire(t, s, c=c):
            pltpu.sync_copy(idx3.at[wid, t], ivs[s])
            for j in range(3):
                pltpu.async_copy(tables[c * 3 + j].at[ivs[s].at[j]],
                                 bufs[s][j], sems[s])

        def _drain(s, c=c):
            for j in range(3):
                pltpu.make_async_copy(tables[c * 3 + j].at[ivs[s].at[j]],
                                      bufs[s][j], sems[s]).wait()

        def _drain_scatter(s, c=c):
            pltpu.make_async_copy(zbs[s], sacc.at[ivs[s].at[c]],
                                  sem_sc).wait()

        def _consume(s, c=c):
            bj = bufs[s]
            scale_l, shift_l = scale_vs, shift_vs

            @plsc.parallel_loop(0, C, unroll=4)
            def _row(r):
                for v in range(8):
                    sl = pl.ds(v * 16, 16)
                    y = bj[0][r, sl] + bj[1][r, sl] + bj[2][r, sl]
                    zbs[s][r, sl] = jnp.maximum(
                        y * scale_l[v] + shift_l[v], 0.0)

            pltpu.async_copy(zbs[s], sacc.at[ivs[s].at[c]], sem_sc, add=True)

        _fire(0, 0)

        def _pair(p, carry):
            for par in (0, 1):
                t = 2 * p + par

                @pl.when(t >= 1)
                def _():
                    _drain_scatter(1 - par)

                @pl.when(t + 1 < NCH)
                def _():
                    _fire(t + 1, 1 - par)

                _drain(par)
                _consume(par)
            return carry

        lax.fori_loop(0, NCH // 2, _pair, 0)
        _drain_scatter(1)
        plsc.subcore_barrier()
        pltpu.sync_copy(sacc.at[pl.ds(sid * RPT, RPT)],
                        s_out.at[cid, c, pl.ds(sid * RPT, RPT)])


R_PRE = 2048


def _tc_pre_body(x_ref, w_ref, b_ref, *out_refs):
    i = pl.program_id(0)
    acc = jnp.dot(x_ref[...], w_ref[...], preferred_element_type=_f32)
    rows = i * R_PRE + lax.broadcasted_iota(_i32, (R_PRE, 1), 0)
    mask = (rows < N).astype(_f32)
    masked = []
    for t in range(9):
        masked.append((acc[:, t * D:(t + 1) * D] + b_ref[t, :][None, :]) * mask)
        out_refs[t][...] = masked[t]
    for j in range(3):
        out_refs[9 + j][...] = jnp.concatenate(
            [masked[0 * 3 + j], masked[1 * 3 + j], masked[2 * 3 + j]], axis=1)


_tc_pre = pl.pallas_call(
    _tc_pre_body,
    grid=(N_PAD // R_PRE,),
    in_specs=[
        pl.BlockSpec((R_PRE, D), lambda i: (i, 0)),
        pl.BlockSpec((D, 9 * D), lambda i: (0, 0)),
        pl.BlockSpec((9, D), lambda i: (0, 0)),
    ],
    out_specs=[pl.BlockSpec((R_PRE, D), lambda i: (i, 0)) for _ in range(9)]
    + [pl.BlockSpec((R_PRE, 3 * D), lambda i: (i, 0)) for _ in range(3)],
    out_shape=[jax.ShapeDtypeStruct((N_PAD, D), _f32) for _ in range(9)]
    + [jax.ShapeDtypeStruct((N_PAD, 3 * D), _f32) for _ in range(3)],
)


def _tc_mid_body(x_ref, hist_ref, ssq_ref, w_ref, b_ref, g_ref, beta_ref, out_ref):
    hist = hist_ref[...]
    xv = x_ref[...]
    s_list = []
    for j in range(3):
        cnt = hist[0, :N, j] + hist[1, :N, j]
        s_list.append(jnp.dot(cnt, xv, preferred_element_type=_f32))
    ssq_tot = jnp.sum(ssq_ref[...], axis=0)
    ef = float(E)
    for c in range(3):
        se = b_ref[c * 3, :] * ef
        for j in range(3):
            t = c * 3 + j
            se = se + jnp.dot(s_list[j], w_ref[:, t * D:(t + 1) * D],
                              preferred_element_type=_f32)
        mu = se / ef
        var = ssq_tot[c] / ef - mu * mu
        scale = g_ref[c, :] / jnp.sqrt(var + EPS)
        shift = beta_ref[c, :] - mu * scale
        out_ref[c, 0, :] = scale
        out_ref[c, 1, :] = shift


_tc_mid = pl.pallas_call(
    _tc_mid_body,
    out_shape=jax.ShapeDtypeStruct((3, 2, D), _f32),
)


R_POST = 2000


def _tc_post1_body(s_ref, hist_ref, w_ref, b_ref, u_ref, stats_ref, acc_ref):
    i = pl.program_id(0)
    sv = s_ref[...]
    hist = hist_ref[...]
    m = jnp.zeros((R_POST, D), _f32)
    for c in range(3):
        cnt = hist[0, :, c] + hist[1, :, c]
        m = m + (sv[0, c] + sv[1, c]) / jnp.maximum(cnt, 1.0)[:, None]
    u = jnp.dot(m, w_ref[...], preferred_element_type=_f32) + b_ref[...][0][None, :]
    u_ref[...] = u

    @pl.when(i == 0)
    def _():
        acc_ref[...] = jnp.zeros((2, D), _f32)

    acc_ref[0, :] += jnp.sum(u, axis=0)
    acc_ref[1, :] += jnp.sum(u * u, axis=0)

    @pl.when(i == pl.num_programs(0) - 1)
    def _():
        stats_ref[...] = acc_ref[...]


_tc_post1 = pl.pallas_call(
    _tc_post1_body,
    grid=(N // R_POST,),
    in_specs=[
        pl.BlockSpec((2, 3, R_POST, D), lambda i: (0, 0, i, 0)),
        pl.BlockSpec((2, R_POST, D), lambda i: (0, i, 0)),
        pl.BlockSpec((D, D), lambda i: (0, 0)),
        pl.BlockSpec((1, D), lambda i: (0, 0)),
    ],
    out_specs=[
        pl.BlockSpec((R_POST, D), lambda i: (i, 0)),
        pl.BlockSpec((2, D), lambda i: (0, 0)),
    ],
    out_shape=[
        jax.ShapeDtypeStruct((N, D), _f32),
        jax.ShapeDtypeStruct((2, D), _f32),
    ],
    scratch_shapes=[pltpu.VMEM((2, D), _f32)],
)


def _tc_post2_body(u_ref, stats_ref, g_ref, beta_ref, x_ref, o_ref):
    st = stats_ref[...]
    nf = float(N)
    mu = st[0] / nf
    var = st[1] / nf - mu * mu
    scale = g_ref[...][0] / jnp.sqrt(var + EPS)
    shift = beta_ref[...][0] - mu * scale
    o_ref[...] = x_ref[...] + jnp.maximum(
        u_ref[...] * scale[None, :] + shift[None, :], 0.0)


_tc_post2 = pl.pallas_call(
    _tc_post2_body,
    grid=(N // R_POST,),
    in_specs=[
        pl.BlockSpec((R_POST, D), lambda i: (i, 0)),
        pl.BlockSpec((2, D), lambda i: (0, 0)),
        pl.BlockSpec((1, D), lambda i: (0, 0)),
        pl.BlockSpec((1, D), lambda i: (0, 0)),
        pl.BlockSpec((R_POST, D), lambda i: (i, 0)),
    ],
    out_specs=pl.BlockSpec((R_POST, D), lambda i: (i, 0)),
    out_shape=jax.ShapeDtypeStruct((N, D), _f32),
)


def kernel(x, term_walk_index, WT, bT, gT, btT, WM, bM, gM, btM,
           WB, bB, gB, btB, Wtw, btw, gtw, bttw):
    idx = term_walk_index.astype(_i32)
    idx = jnp.concatenate(
        [idx, jnp.full((3, E_PAD - E), N, _i32)], axis=1)
    idx3 = jnp.transpose(idx.reshape(3, NW, NCH, C), (1, 2, 0, 3))
    zcol = jnp.zeros((D,), _f32)
    hist = _sc_hist(idx3)

    for i in range(K):
        wall = jnp.concatenate(
            [WT[i][0:D], WT[i][D:2 * D], WT[i][2 * D:3 * D],
             WM[i][0:D], WM[i][D:2 * D], WM[i][2 * D:3 * D],
             WB[i][0:D], WB[i][D:2 * D], WB[i][2 * D:3 * D]], axis=1)
        ball = jnp.stack([bT[i], zcol, zcol, bM[i], zcol, zcol,
                          bB[i], zcol, zcol])
        gstack = jnp.stack([gT[i], gM[i], gB[i]])
        betastack = jnp.stack([btT[i], btM[i], btB[i]])
        x_pad = jnp.concatenate([x, jnp.zeros((N_PAD - N, D), _f32)])

        outs = _tc_pre(x_pad, wall, ball)
        tables, gtabs = outs[:9], outs[9:]
        ssq = _sc_pass_a(*gtabs, idx3)
        ss = _tc_mid(x, hist, ssq, wall, ball, gstack, betastack)
        s_part = _sc_pass_b(*tables, idx3, ss)
        u, stats = _tc_post1(s_part, hist, Wtw[i], btw[i].reshape(1, D))
        x = _tc_post2(u, stats, gtw[i].reshape(1, D),
                      bttw[i].reshape(1, D), x)
    return x


# speedup vs baseline: 1.0176x; 1.0176x over previous
import functools

import jax
import jax.numpy as jnp
from jax import lax
from jax.experimental import pallas as pl
from jax.experimental.pallas import tpu as pltpu
from jax.experimental.pallas import tpu_sc as plsc

N = 10000
E = 320000
D = 128
K = 2
EPS = 1e-5

N_PAD = 10240
NW = 32
C = 40
EPW = 10080
NCH = EPW // C
CA = 48
NCH_A = EPW // CA
E_PAD = NW * EPW
RPT = N_PAD // 16

_mesh = plsc.VectorSubcoreMesh(core_axis_name="c", subcore_axis_name="s")

_f32 = jnp.float32
_i32 = jnp.int32


@functools.partial(
    pl.kernel,
    mesh=_mesh,
    out_type=jax.ShapeDtypeStruct((2, N_PAD, D), _f32),
    scratch_types=[
        pltpu.VMEM((3, C), _i32),
        pltpu.VMEM((C, D), _f32),
        pltpu.VMEM((C, D), _f32),
        pltpu.VMEM((C, D), _f32),
        pltpu.VMEM_SHARED((N_PAD, D), _f32),
        pltpu.SemaphoreType.DMA,
    ],
)
def _sc_hist(idx3, hist_out, iv, ones0, ones1, ones2, hacc, sem):
    ones = (ones0, ones1, ones2)
    cid = lax.axis_index("c")
    sid = lax.axis_index("s")
    wid = sid * 2 + cid

    z16 = jnp.zeros((16,), _f32)

    def _fill(r, carry):
        for j in range(3):
            for v in range(8):
                ones[j][r, pl.ds(v * 16, 16)] = z16
        return carry

    lax.fori_loop(0, C, _fill, 0)

    def _zero_acc(q, carry):
        pltpu.sync_copy(ones0, hacc.at[pl.ds(sid * RPT + q * C, C)])
        return carry

    lax.fori_loop(0, RPT // C, _zero_acc, 0)

    for j in range(3):
        onerow = jnp.where(lax.iota(_i32, 16) == j, 1.0, 0.0).astype(_f32)

        def _fill_ones(r, carry, j=j, onerow=onerow):
            ones[j][r, pl.ds(0, 16)] = onerow
            return carry

        lax.fori_loop(0, C, _fill_ones, 0)

    plsc.subcore_barrier()

    def _chunk(t, carry):
        pltpu.sync_copy(idx3.at[wid, t], iv)
        for j in range(3):
            pltpu.sync_copy(ones[j], hacc.at[iv.at[j]], add=True)
        return carry

    lax.fori_loop(0, NCH, _chunk, 0)

    plsc.subcore_barrier()
    pltpu.sync_copy(hacc.at[pl.ds(sid * RPT, RPT)],
                    hist_out.at[cid, pl.ds(sid * RPT, RPT)])


@functools.partial(
    pl.kernel,
    mesh=_mesh,
    out_type=jax.ShapeDtypeStruct((NW, 3, D), _f32),
    scratch_types=[
        pltpu.VMEM((3, CA), _i32),
        pltpu.VMEM((3, CA), _i32),
    ] + [pltpu.VMEM((CA, 3 * D), _f32) for _ in range(6)] + [
        pltpu.VMEM((3, D), _f32),
        pltpu.SemaphoreType.DMA,
        pltpu.SemaphoreType.DMA,
    ],
)
def _sc_pass_a(g0, g1, g2, idx3, ssq_out,
               iv_a, iv_b, ba0, ba1, ba2, bb0, bb1, bb2,
               ssq_v, sem_a, sem_b):
    tabs = (g0, g1, g2)
    ivs = (iv_a, iv_b)
    bufs = ((ba0, ba1, ba2), (bb0, bb1, bb2))
    sems = (sem_a, sem_b)
    cid = lax.axis_index("c")
    sid = lax.axis_index("s")
    wid = sid * 2 + cid

    z16 = jnp.zeros((16,), _f32)

    def _fire(t, s):
        pltpu.sync_copy(idx3.at[wid, t], ivs[s])
        for j in range(3):
            pltpu.async_copy(tabs[j].at[ivs[s].at[j]], bufs[s][j], sems[s])

    def _drain(s):
        for j in range(3):
            pltpu.make_async_copy(tabs[j].at[ivs[s].at[j]], bufs[s][j],
                                  sems[s]).wait()

    def _consume(s, acc):
        bj = bufs[s]

        def _row(r, a):
            out = []
            for c in range(3):
                for v in range(8):
                    sl = pl.ds(c * D + v * 16, 16)
                    y = bj[0][r, sl] + bj[1][r, sl] + bj[2][r, sl]
                    out.append(a[c * 8 + v] + y * y)
            return tuple(out)

        return lax.fori_loop(0, CA, _row, acc)

    _fire(0, 0)
    init = tuple(z16 for _ in range(24))

    def _pair(p, acc):
        for par in (0, 1):
            t = 2 * p + par

            @pl.when(t + 1 < NCH_A)
            def _():
                _fire(t + 1, 1 - par)

            _drain(par)
            acc = _consume(par, acc)
        return acc

    acc = lax.fori_loop(0, NCH_A // 2, _pair, init)
    for c in range(3):
        for v in range(8):
            ssq_v[c, pl.ds(v * 16, 16)] = acc[c * 8 + v]
    pltpu.sync_copy(ssq_v, ssq_out.at[wid])


@functools.partial(
    pl.kernel,
    mesh=_mesh,
    out_type=jax.ShapeDtypeStruct((2, 3, N_PAD, D), _f32),
    scratch_types=[
        pltpu.VMEM((3, C), _i32),
        pltpu.VMEM((3, C), _i32),
    ] + [pltpu.VMEM((C, D), _f32) for _ in range(6)] + [
        pltpu.VMEM((C, D), _f32),
        pltpu.VMEM((C, D), _f32),
        pltpu.VMEM((3, 2, D), _f32),
        pltpu.VMEM_SHARED((N_PAD, D), _f32),
        pltpu.SemaphoreType.DMA,
        pltpu.SemaphoreType.DMA,
        pltpu.SemaphoreType.DMA,
    ],
)
def _sc_pass_b(t0, t1, t2, t3, t4, t5, t6, t7, t8, idx3, ss,
               s_out,
               iv_a, iv_b, ba0, ba1, ba2, bb0, bb1, bb2,
               zb0, zb1, ss_v, sacc, sem_a, sem_b, sem_sc):
    tables = (t0, t1, t2, t3, t4, t5, t6, t7, t8)
    ivs = (iv_a, iv_b)
    bufs = ((ba0, ba1, ba2), (bb0, bb1, bb2))
    zbs = (zb0, zb1)
    sems = (sem_a, sem_b)
    cid = lax.axis_index("c")
    sid = lax.axis_index("s")
    wid = sid * 2 + cid

    z16 = jnp.zeros((16,), _f32)

    pltpu.sync_copy(ss, ss_v)

    def _fill_zero(r, carry):
        for v in range(8):
            zb0[r, pl.ds(v * 16, 16)] = z16
        return carry

    lax.fori_loop(0, C, _fill_zero, 0)

    for c in range(3):
        scale_vs = [ss_v[c, 0, pl.ds(v * 16, 16)] for v in range(8)]
        shift_vs = [ss_v[c, 1, pl.ds(v * 16, 16)] for v in range(8)]

        if c > 0:
            lax.fori_loop(0, C, _fill_zero, 0)

        def _zero_acc(q, carry):
            pltpu.sync_copy(zb0, sacc.at[pl.ds(sid * RPT + q * C, C)])
            return carry

        lax.fori_loop(0, RPT // C, _zero_acc, 0)
        plsc.subcore_barrier()

        def _fire(t, s, c=c):
            pltpu.sync_copy(idx3.at[wid, t], ivs[s])
            for j in range(3):
                pltpu.async_copy(tables[c * 3 + j].at[ivs[s].at[j]],
                                 bufs[s][j], sems[s])

        def _drain(s, c=c):
            for j in range(3):
                pltpu.make_async_copy(tables[c * 3 + j].at[ivs[s].at[j]],
                                      bufs[s][j], sems[s]).wait()

        def _drain_scatter(s, c=c):
            pltpu.make_async_copy(zbs[s], sacc.at[ivs[s].at[c]],
                                  sem_sc).wait()

        def _consume(s, c=c):
            bj = bufs[s]
            scale_l, shift_l = scale_vs, shift_vs

            def _row(r, carry):
                for v in range(8):
                    sl = pl.ds(v * 16, 16)
                    y = bj[0][r, sl] + bj[1][r, sl] + bj[2][r, sl]
                    zbs[s][r, sl] = jnp.maximum(
                        y * scale_l[v] + shift_l[v], 0.0)
                return carry

            lax.fori_loop(0, C, _row, 0)
            pltpu.async_copy(zbs[s], sacc.at[ivs[s].at[c]], sem_sc, add=True)

        _fire(0, 0)

        def _pair(p, carry):
            for par in (0, 1):
                t = 2 * p + par

                @pl.when(t >= 1)
                def _():
                    _drain_scatter(1 - par)

                @pl.when(t + 1 < NCH)
                def _():
                    _fire(t + 1, 1 - par)

                _drain(par)
                _consume(par)
            return carry

        lax.fori_loop(0, NCH // 2, _pair, 0)
        _drain_scatter(1)
        plsc.subcore_barrier()
        pltpu.sync_copy(sacc.at[pl.ds(sid * RPT, RPT)],
                        s_out.at[cid, c, pl.ds(sid * RPT, RPT)])


R_PRE = 2048


def _tc_pre_body(x_ref, w_ref, b_ref, *out_refs):
    i = pl.program_id(0)
    acc = jnp.dot(x_ref[...], w_ref[...], preferred_element_type=_f32)
    rows = i * R_PRE + lax.broadcasted_iota(_i32, (R_PRE, 1), 0)
    mask = (rows < N).astype(_f32)
    masked = []
    for t in range(9):
        masked.append((acc[:, t * D:(t + 1) * D] + b_ref[t, :][None, :]) * mask)
        out_refs[t][...] = masked[t]
    for j in range(3):
        out_refs[9 + j][...] = jnp.concatenate(
            [masked[0 * 3 + j], masked[1 * 3 + j], masked[2 * 3 + j]], axis=1)


_tc_pre = pl.pallas_call(
    _tc_pre_body,
    grid=(N_PAD // R_PRE,),
    in_specs=[
        pl.BlockSpec((R_PRE, D), lambda i: (i, 0)),
        pl.BlockSpec((D, 9 * D), lambda i: (0, 0)),
        pl.BlockSpec((9, D), lambda i: (0, 0)),
    ],
    out_specs=[pl.BlockSpec((R_PRE, D), lambda i: (i, 0)) for _ in range(9)]
    + [pl.BlockSpec((R_PRE, 3 * D), lambda i: (i, 0)) for _ in range(3)],
    out_shape=[jax.ShapeDtypeStruct((N_PAD, D), _f32) for _ in range(9)]
    + [jax.ShapeDtypeStruct((N_PAD, 3 * D), _f32) for _ in range(3)],
)


def _tc_mid_body(x_ref, hist_ref, ssq_ref, w_ref, b_ref, g_ref, beta_ref, out_ref):
    hist = hist_ref[...]
    xv = x_ref[...]
    s_list = []
    for j in range(3):
        cnt = hist[0, :N, j] + hist[1, :N, j]
        s_list.append(jnp.dot(cnt, xv, preferred_element_type=_f32))
    ssq_tot = jnp.sum(ssq_ref[...], axis=0)
    ef = float(E)
    for c in range(3):
        se = b_ref[c * 3, :] * ef
        for j in range(3):
            t = c * 3 + j
            se = se + jnp.dot(s_list[j], w_ref[:, t * D:(t + 1) * D],
                              preferred_element_type=_f32)
        mu = se / ef
        var = ssq_tot[c] / ef - mu * mu
        scale = g_ref[c, :] / jnp.sqrt(var + EPS)
        shift = beta_ref[c, :] - mu * scale
        out_ref[c, 0, :] = scale
        out_ref[c, 1, :] = shift


_tc_mid = pl.pallas_call(
    _tc_mid_body,
    out_shape=jax.ShapeDtypeStruct((3, 2, D), _f32),
)


R_POST = 2000


def _tc_post1_body(s_ref, hist_ref, w_ref, b_ref, u_ref, stats_ref, acc_ref):
    i = pl.program_id(0)
    sv = s_ref[...]
    hist = hist_ref[...]
    m = jnp.zeros((R_POST, D), _f32)
    for c in range(3):
        cnt = hist[0, :, c] + hist[1, :, c]
        m = m + (sv[0, c] + sv[1, c]) / jnp.maximum(cnt, 1.0)[:, None]
    u = jnp.dot(m, w_ref[...], preferred_element_type=_f32) + b_ref[...][0][None, :]
    u_ref[...] = u

    @pl.when(i == 0)
    def _():
        acc_ref[...] = jnp.zeros((2, D), _f32)

    acc_ref[0, :] += jnp.sum(u, axis=0)
    acc_ref[1, :] += jnp.sum(u * u, axis=0)

    @pl.when(i == pl.num_programs(0) - 1)
    def _():
        stats_ref[...] = acc_ref[...]


_tc_post1 = pl.pallas_call(
    _tc_post1_body,
    grid=(N // R_POST,),
    in_specs=[
        pl.BlockSpec((2, 3, R_POST, D), lambda i: (0, 0, i, 0)),
        pl.BlockSpec((2, R_POST, D), lambda i: (0, i, 0)),
        pl.BlockSpec((D, D), lambda i: (0, 0)),
        pl.BlockSpec((1, D), lambda i: (0, 0)),
    ],
    out_specs=[
        pl.BlockSpec((R_POST, D), lambda i: (i, 0)),
        pl.BlockSpec((2, D), lambda i: (0, 0)),
    ],
    out_shape=[
        jax.ShapeDtypeStruct((N, D), _f32),
        jax.ShapeDtypeStruct((2, D), _f32),
    ],
    scratch_shapes=[pltpu.VMEM((2, D), _f32)],
)


def _tc_post2_body(u_ref, stats_ref, g_ref, beta_ref, x_ref, o_ref):
    st = stats_ref[...]
    nf = float(N)
    mu = st[0] / nf
    var = st[1] / nf - mu * mu
    scale = g_ref[...][0] / jnp.sqrt(var + EPS)
    shift = beta_ref[...][0] - mu * scale
    o_ref[...] = x_ref[...] + jnp.maximum(
        u_ref[...] * scale[None, :] + shift[None, :], 0.0)


_tc_post2 = pl.pallas_call(
    _tc_post2_body,
    grid=(N // R_POST,),
    in_specs=[
        pl.BlockSpec((R_POST, D), lambda i: (i, 0)),
        pl.BlockSpec((2, D), lambda i: (0, 0)),
        pl.BlockSpec((1, D), lambda i: (0, 0)),
        pl.BlockSpec((1, D), lambda i: (0, 0)),
        pl.BlockSpec((R_POST, D), lambda i: (i, 0)),
    ],
    out_specs=pl.BlockSpec((R_POST, D), lambda i: (i, 0)),
    out_shape=jax.ShapeDtypeStruct((N, D), _f32),
)


def kernel(x, term_walk_index, WT, bT, gT, btT, WM, bM, gM, btM,
           WB, bB, gB, btB, Wtw, btw, gtw, bttw):
    idx = term_walk_index.astype(_i32)
    idx = jnp.concatenate(
        [idx, jnp.full((3, E_PAD - E), N, _i32)], axis=1)
    idx3 = jnp.transpose(idx.reshape(3, NW, NCH, C), (1, 2, 0, 3))
    idx3a = jnp.transpose(idx.reshape(3, NW, NCH_A, CA), (1, 2, 0, 3))
    zcol = jnp.zeros((D,), _f32)
    hist = _sc_hist(idx3)

    for i in range(K):
        wall = jnp.concatenate(
            [WT[i][0:D], WT[i][D:2 * D], WT[i][2 * D:3 * D],
             WM[i][0:D], WM[i][D:2 * D], WM[i][2 * D:3 * D],
             WB[i][0:D], WB[i][D:2 * D], WB[i][2 * D:3 * D]], axis=1)
        ball = jnp.stack([bT[i], zcol, zcol, bM[i], zcol, zcol,
                          bB[i], zcol, zcol])
        gstack = jnp.stack([gT[i], gM[i], gB[i]])
        betastack = jnp.stack([btT[i], btM[i], btB[i]])
        x_pad = jnp.concatenate([x, jnp.zeros((N_PAD - N, D), _f32)])

        outs = _tc_pre(x_pad, wall, ball)
        tables, gtabs = outs[:9], outs[9:]
        ssq = _sc_pass_a(*gtabs, idx3a)
        ss = _tc_mid(x, hist, ssq, wall, ball, gstack, betastack)
        s_part = _sc_pass_b(*tables, idx3, ss)
        u, stats = _tc_post1(s_part, hist, Wtw[i], btw[i].reshape(1, D))
        x = _tc_post2(u, stats, gtw[i].reshape(1, D),
                      bttw[i].reshape(1, D), x)
    return x
